# Initial kernel scaffold; baseline (speedup 1.0000x reference)
import functools

import jax
import jax.numpy as jnp
from jax import lax
from jax.experimental import pallas as pl
from jax.experimental.pallas import tpu as pltpu
from jax.experimental.pallas import tpu_sc as plsc

N = 10000
D = 128
E = 320000

NC = 2
NS = 16
NW = NC * NS
CHUNK = 128
EPW = 10240
NCHUNK = EPW // CHUNK
EP = NW * EPW
RPS = N // NS

BLK = 1000

_mesh = plsc.VectorSubcoreMesh(core_axis_name="c", subcore_axis_name="s")


@functools.partial(
    pl.kernel,
    out_type=jax.ShapeDtypeStruct((NC, N, 16), jnp.float32),
    mesh=_mesh,
    scratch_types=[
        pltpu.VMEM_SHARED((N, 16), jnp.float32),
        pltpu.VMEM((CHUNK,), jnp.int32),
        pltpu.VMEM((CHUNK,), jnp.float32),
        pltpu.VMEM((CHUNK, 16), jnp.float32),
    ],
)
def _deg_kernel(dst_hbm, ew_hbm, out_hbm, deg_sh, dst_v, ew_v, bc_v):
    c = lax.axis_index("c")
    s = lax.axis_index("s")
    wid = c * NS + s
    zero16 = jnp.zeros((16,), jnp.float32)

    @pl.loop(0, CHUNK)
    def _(r):
        bc_v[r, :] = zero16

    base = s * RPS

    @pl.loop(0, 4)
    def _(t):
        pltpu.sync_copy(bc_v, deg_sh.at[pl.ds(base + t * CHUNK, CHUNK)])

    pltpu.sync_copy(bc_v.at[pl.ds(0, RPS - 4 * CHUNK)],
                    deg_sh.at[pl.ds(base + 4 * CHUNK, RPS - 4 * CHUNK)])
    plsc.subcore_barrier()

    @pl.loop(0, NCHUNK)
    def _(k):
        ebase = wid * EPW + k * CHUNK
        pltpu.sync_copy(dst_hbm.at[pl.ds(ebase, CHUNK)], dst_v)
        pltpu.sync_copy(ew_hbm.at[pl.ds(ebase, CHUNK)], ew_v)

        @pl.loop(0, CHUNK // 16)
        def _(g):
            ew16 = ew_v[pl.ds(pl.multiple_of(g * 16, 16), 16)]
            for j in range(16):
                sp = jnp.take(ew16, jnp.full((16,), j, jnp.int32),
                              mode="promise_in_bounds")
                bc_v[g * 16 + j, :] = sp

        pltpu.sync_copy(bc_v, deg_sh.at[dst_v], add=True)

    plsc.subcore_barrier()
    pltpu.sync_copy(deg_sh.at[pl.ds(base, RPS)],
                    out_hbm.at[c, pl.ds(base, RPS)])


@functools.partial(
    pl.kernel,
    out_type=jax.ShapeDtypeStruct((NC, N, D), jnp.float32),
    mesh=_mesh,
    scratch_types=[
        pltpu.VMEM_SHARED((N, D), jnp.float32),
        pltpu.VMEM((CHUNK,), jnp.int32),
        pltpu.VMEM((CHUNK,), jnp.int32),
        pltpu.VMEM((CHUNK,), jnp.float32),
        pltpu.VMEM((CHUNK, D), jnp.float32),
        pltpu.SemaphoreType.DMA,
    ],
)
def _agg_kernel(h_hbm, src_hbm, dst_hbm, ew_hbm, out_hbm,
                acc_sh, src_v, dst_v, ew_v, rows_v, sem):
    c = lax.axis_index("c")
    s = lax.axis_index("s")
    wid = c * NS + s
    zero16 = jnp.zeros((16,), jnp.float32)

    @pl.loop(0, CHUNK)
    def _(r):
        for q in range(D // 16):
            rows_v[r, pl.ds(q * 16, 16)] = zero16

    base = s * RPS

    @pl.loop(0, 4)
    def _(t):
        pltpu.sync_copy(rows_v, acc_sh.at[pl.ds(base + t * CHUNK, CHUNK)])

    pltpu.sync_copy(rows_v.at[pl.ds(0, RPS - 4 * CHUNK)],
                    acc_sh.at[pl.ds(base + 4 * CHUNK, RPS - 4 * CHUNK)])
    plsc.subcore_barrier()

    @pl.loop(0, NCHUNK)
    def _(k):
        ebase = wid * EPW + k * CHUNK
        pltpu.sync_copy(src_hbm.at[pl.ds(ebase, CHUNK)], src_v)
        pltpu.sync_copy(dst_hbm.at[pl.ds(ebase, CHUNK)], dst_v)
        pltpu.sync_copy(ew_hbm.at[pl.ds(ebase, CHUNK)], ew_v)
        pltpu.async_copy(h_hbm.at[src_v], rows_v, sem).wait()

        @pl.loop(0, CHUNK // 16)
        def _(g):
            ew16 = ew_v[pl.ds(pl.multiple_of(g * 16, 16), 16)]
            for j in range(16):
                sp = jnp.take(ew16, jnp.full((16,), j, jnp.int32),
                              mode="promise_in_bounds")
                for q in range(D // 16):
                    sl = (g * 16 + j, pl.ds(q * 16, 16))
                    rows_v[sl] = rows_v[sl] * sp

        pltpu.sync_copy(rows_v, acc_sh.at[dst_v], add=True)

    plsc.subcore_barrier()
    pltpu.sync_copy(acc_sh.at[pl.ds(base, RPS)],
                    out_hbm.at[c, pl.ds(base, RPS)])


def _mm_body(x_ref, w_ref, o_ref):
    o_ref[...] = jnp.dot(x_ref[...], w_ref[...],
                         preferred_element_type=jnp.float32)


def _mm(x, w):
    return pl.pallas_call(
        _mm_body,
        grid=(N // BLK,),
        in_specs=[pl.BlockSpec((BLK, D), lambda i: (i, 0)),
                  pl.BlockSpec((D, D), lambda i: (0, 0))],
        out_specs=pl.BlockSpec((BLK, D), lambda i: (i, 0)),
        out_shape=jax.ShapeDtypeStruct((N, D), jnp.float32),
    )(x, w)


def _prep_body(degp_ref, h1_ref, dis_ref, h1p_ref):
    deg16 = degp_ref[0] + degp_ref[1] + 1.0
    dis16 = lax.rsqrt(deg16)
    dis = jnp.broadcast_to(dis16[:, :1], (BLK, D))
    dis_ref[...] = dis
    h1p_ref[...] = h1_ref[...] * dis


def _prep(degp, h1):
    return pl.pallas_call(
        _prep_body,
        grid=(N // BLK,),
        in_specs=[pl.BlockSpec((NC, BLK, 16), lambda i: (0, i, 0)),
                  pl.BlockSpec((BLK, D), lambda i: (i, 0))],
        out_specs=[pl.BlockSpec((BLK, D), lambda i: (i, 0)),
                   pl.BlockSpec((BLK, D), lambda i: (i, 0))],
        out_shape=[jax.ShapeDtypeStruct((N, D), jnp.float32),
                   jax.ShapeDtypeStruct((N, D), jnp.float32)],
    )(degp, h1)


def _mid_body(p_ref, dis_ref, h1_ref, b_ref, w_ref, h2_ref, h2p_ref):
    dis = dis_ref[...]
    o1 = dis * (p_ref[0] + p_ref[1]) + dis * dis * h1_ref[...] + b_ref[...]
    r = jnp.maximum(o1, 0.0)
    h2 = jnp.dot(r, w_ref[...], preferred_element_type=jnp.float32)
    h2_ref[...] = h2
    h2p_ref[...] = h2 * dis


def _mid(p, dis, h1, b1, w2):
    return pl.pallas_call(
        _mid_body,
        grid=(N // BLK,),
        in_specs=[pl.BlockSpec((NC, BLK, D), lambda i: (0, i, 0)),
                  pl.BlockSpec((BLK, D), lambda i: (i, 0)),
                  pl.BlockSpec((BLK, D), lambda i: (i, 0)),
                  pl.BlockSpec((1, D), lambda i: (0, 0)),
                  pl.BlockSpec((D, D), lambda i: (0, 0))],
        out_specs=[pl.BlockSpec((BLK, D), lambda i: (i, 0)),
                   pl.BlockSpec((BLK, D), lambda i: (i, 0))],
        out_shape=[jax.ShapeDtypeStruct((N, D), jnp.float32),
                   jax.ShapeDtypeStruct((N, D), jnp.float32)],
    )(p, dis, h1, b1, w2)


def _fin_body(p_ref, dis_ref, h2_ref, b_ref, o_ref):
    dis = dis_ref[...]
    o2 = dis * (p_ref[0] + p_ref[1]) + dis * dis * h2_ref[...] + b_ref[...]
    nrm = jnp.sqrt(jnp.sum(o2 * o2, axis=1, keepdims=True))
    o_ref[...] = o2 / jnp.maximum(nrm, 1e-12)


def _fin(p, dis, h2, b2):
    return pl.pallas_call(
        _fin_body,
        grid=(N // BLK,),
        in_specs=[pl.BlockSpec((NC, BLK, D), lambda i: (0, i, 0)),
                  pl.BlockSpec((BLK, D), lambda i: (i, 0)),
                  pl.BlockSpec((BLK, D), lambda i: (i, 0)),
                  pl.BlockSpec((1, D), lambda i: (0, 0))],
        out_specs=pl.BlockSpec((BLK, D), lambda i: (i, 0)),
        out_shape=jax.ShapeDtypeStruct((N, D), jnp.float32),
    )(p, dis, h2, b2)


def kernel(x, edge_index, edge_weight, W1, b1, W2, b2):
    src = edge_index[0].astype(jnp.int32)
    dst = edge_index[1].astype(jnp.int32)
    ew = edge_weight.astype(jnp.float32)
    pad = EP - E
    src = jnp.concatenate([src, jnp.zeros((pad,), jnp.int32)])
    dst = jnp.concatenate([dst, jnp.zeros((pad,), jnp.int32)])
    ew = jnp.concatenate([ew, jnp.zeros((pad,), jnp.float32)])
    b1 = b1.reshape(1, D)
    b2 = b2.reshape(1, D)

    degp = _deg_kernel(dst, ew)
    h1 = _mm(x, W1)
    dis, h1p = _prep(degp, h1)
    p1 = _agg_kernel(h1p, src, dst, ew)
    h2, h2p = _mid(p1, dis, h1, b1, W2)
    p2 = _agg_kernel(h2p, src, dst, ew)
    return _fin(p2, dis, h2, b2)


# speedup vs baseline: 6.3654x; 6.3654x over previous
import dataclasses
import functools

import jax
import jax.numpy as jnp
from jax import lax
from jax.experimental import pallas as pl
from jax.experimental.pallas import tpu as pltpu
from jax.experimental.pallas import tpu_sc as plsc

N = 10000
D = 128
E = 320000

NC = 2
NS = 16
NW = NC * NS
CHUNK = 128
EPW = 10240
NCHUNK = EPW // CHUNK
EP = NW * EPW
NP = 10240
RPS = NP // NS

BLK = 1000

_mesh = plsc.VectorSubcoreMesh(core_axis_name="c", subcore_axis_name="s")

_sc_params = pltpu.CompilerParams()
if "needs_layout_passes" in pltpu.CompilerParams.__dataclass_fields__:
    _sc_params = dataclasses.replace(_sc_params, needs_layout_passes=False)

_GATHER_DNUMS = lax.GatherDimensionNumbers(
    offset_dims=(), collapsed_slice_dims=(0,), start_index_map=(0,))


def _splat(vec16, j):
    idx = jnp.full((16, 1), j, jnp.int32)
    return lax.gather(vec16, idx, _GATHER_DNUMS, slice_sizes=(1,),
                      mode=lax.GatherScatterMode.PROMISE_IN_BOUNDS)


@functools.partial(
    pl.kernel,
    out_type=jax.ShapeDtypeStruct((NC, NP, D), jnp.float32),
    mesh=_mesh,
    compiler_params=_sc_params,
    scratch_types=[
        pltpu.VMEM_SHARED((NP, D), jnp.float32),
        pltpu.VMEM((CHUNK,), jnp.int32),
        pltpu.VMEM((CHUNK,), jnp.float32),
        pltpu.VMEM((CHUNK, D), jnp.float32),
    ],
)
def _deg_kernel(dst_hbm, ew_hbm, out_hbm, deg_sh, dst_v, ew_v, bc_v):
    c = lax.axis_index("c")
    s = lax.axis_index("s")
    wid = c * NS + s
    zero16 = jnp.zeros((16,), jnp.float32)

    @pl.loop(0, CHUNK)
    def _(r):
        for q in range(D // 16):
            bc_v.at[r, pl.ds(q * 16, 16)][...] = zero16

    base = s * RPS

    @pl.loop(0, RPS // CHUNK)
    def _(t):
        pltpu.sync_copy(bc_v, deg_sh.at[pl.ds(base + t * CHUNK, CHUNK)])
    plsc.subcore_barrier()

    @pl.loop(0, NCHUNK)
    def _(k):
        ebase = wid * EPW + k * CHUNK
        pltpu.sync_copy(dst_hbm.at[pl.ds(ebase, CHUNK)], dst_v)
        pltpu.sync_copy(ew_hbm.at[pl.ds(ebase, CHUNK)], ew_v)

        @pl.loop(0, CHUNK // 16)
        def _(g):
            ew16 = ew_v[pl.ds(pl.multiple_of(g * 16, 16), 16)]
            for j in range(16):
                bc_v.at[g * 16 + j, pl.ds(0, 16)][...] = _splat(ew16, j)

        pltpu.sync_copy(bc_v, deg_sh.at[dst_v], add=True)

    plsc.subcore_barrier()
    pltpu.sync_copy(deg_sh.at[pl.ds(base, RPS)],
                    out_hbm.at[c, pl.ds(base, RPS)])


@functools.partial(
    pl.kernel,
    out_type=jax.ShapeDtypeStruct((NC, NP, D), jnp.float32),
    mesh=_mesh,
    compiler_params=_sc_params,
    scratch_types=[
        pltpu.VMEM_SHARED((NP, D), jnp.float32),
        pltpu.VMEM((CHUNK,), jnp.int32),
        pltpu.VMEM((CHUNK,), jnp.int32),
        pltpu.VMEM((CHUNK,), jnp.float32),
        pltpu.VMEM((CHUNK, D), jnp.float32),
        pltpu.SemaphoreType.DMA,
    ],
)
def _agg_kernel(h_hbm, src_hbm, dst_hbm, ew_hbm, out_hbm,
                acc_sh, src_v, dst_v, ew_v, rows_v, sem):
    c = lax.axis_index("c")
    s = lax.axis_index("s")
    wid = c * NS + s
    zero16 = jnp.zeros((16,), jnp.float32)

    @pl.loop(0, CHUNK)
    def _(r):
        for q in range(D // 16):
            rows_v.at[r, pl.ds(q * 16, 16)][...] = zero16

    base = s * RPS

    @pl.loop(0, RPS // CHUNK)
    def _(t):
        pltpu.sync_copy(rows_v, acc_sh.at[pl.ds(base + t * CHUNK, CHUNK)])
    plsc.subcore_barrier()

    @pl.loop(0, NCHUNK)
    def _(k):
        ebase = wid * EPW + k * CHUNK
        pltpu.sync_copy(src_hbm.at[pl.ds(ebase, CHUNK)], src_v)
        pltpu.sync_copy(dst_hbm.at[pl.ds(ebase, CHUNK)], dst_v)
        pltpu.sync_copy(ew_hbm.at[pl.ds(ebase, CHUNK)], ew_v)
        pltpu.async_copy(h_hbm.at[src_v], rows_v, sem).wait()

        @pl.loop(0, CHUNK // 16)
        def _(g):
            ew16 = ew_v[pl.ds(pl.multiple_of(g * 16, 16), 16)]
            for j in range(16):
                sp = _splat(ew16, j)
                for q in range(D // 16):
                    slot = rows_v.at[g * 16 + j, pl.ds(q * 16, 16)]
                    slot[...] = slot[...] * sp

        pltpu.sync_copy(rows_v, acc_sh.at[dst_v], add=True)

    plsc.subcore_barrier()
    pltpu.sync_copy(acc_sh.at[pl.ds(base, RPS)],
                    out_hbm.at[c, pl.ds(base, RPS)])


def _mm_body(x_ref, w_ref, o_ref):
    o_ref[...] = jnp.dot(x_ref[...], w_ref[...],
                         preferred_element_type=jnp.float32)


def _mm(x, w):
    return pl.pallas_call(
        _mm_body,
        grid=(N // BLK,),
        in_specs=[pl.BlockSpec((BLK, D), lambda i: (i, 0)),
                  pl.BlockSpec((D, D), lambda i: (0, 0))],
        out_specs=pl.BlockSpec((BLK, D), lambda i: (i, 0)),
        out_shape=jax.ShapeDtypeStruct((N, D), jnp.float32),
    )(x, w)


def _prep_body(degp_ref, h1_ref, dis_ref, h1p_ref):
    degc = degp_ref[0, :, :1] + degp_ref[1, :, :1] + 1.0
    disc = lax.rsqrt(degc)
    dis = jnp.broadcast_to(disc, (BLK, D))
    dis_ref[...] = dis
    h1p_ref[...] = h1_ref[...] * dis


def _prep(degp, h1):
    return pl.pallas_call(
        _prep_body,
        grid=(N // BLK,),
        in_specs=[pl.BlockSpec((NC, BLK, D), lambda i: (0, i, 0)),
                  pl.BlockSpec((BLK, D), lambda i: (i, 0))],
        out_specs=[pl.BlockSpec((BLK, D), lambda i: (i, 0)),
                   pl.BlockSpec((BLK, D), lambda i: (i, 0))],
        out_shape=[jax.ShapeDtypeStruct((N, D), jnp.float32),
                   jax.ShapeDtypeStruct((N, D), jnp.float32)],
    )(degp, h1)


def _mid_body(p_ref, dis_ref, h1_ref, b_ref, w_ref, h2_ref, h2p_ref):
    dis = dis_ref[...]
    o1 = dis * (p_ref[0] + p_ref[1]) + dis * dis * h1_ref[...] + b_ref[...]
    r = jnp.maximum(o1, 0.0)
    h2 = jnp.dot(r, w_ref[...], preferred_element_type=jnp.float32)
    h2_ref[...] = h2
    h2p_ref[...] = h2 * dis


def _mid(p, dis, h1, b1, w2):
    return pl.pallas_call(
        _mid_body,
        grid=(N // BLK,),
        in_specs=[pl.BlockSpec((NC, BLK, D), lambda i: (0, i, 0)),
                  pl.BlockSpec((BLK, D), lambda i: (i, 0)),
                  pl.BlockSpec((BLK, D), lambda i: (i, 0)),
                  pl.BlockSpec((1, D), lambda i: (0, 0)),
                  pl.BlockSpec((D, D), lambda i: (0, 0))],
        out_specs=[pl.BlockSpec((BLK, D), lambda i: (i, 0)),
                   pl.BlockSpec((BLK, D), lambda i: (i, 0))],
        out_shape=[jax.ShapeDtypeStruct((N, D), jnp.float32),
                   jax.ShapeDtypeStruct((N, D), jnp.float32)],
    )(p, dis, h1, b1, w2)


def _fin_body(p_ref, dis_ref, h2_ref, b_ref, o_ref):
    dis = dis_ref[...]
    o2 = dis * (p_ref[0] + p_ref[1]) + dis * dis * h2_ref[...] + b_ref[...]
    nrm = jnp.sqrt(jnp.sum(o2 * o2, axis=1, keepdims=True))
    o_ref[...] = o2 / jnp.maximum(nrm, 1e-12)


def _fin(p, dis, h2, b2):
    return pl.pallas_call(
        _fin_body,
        grid=(N // BLK,),
        in_specs=[pl.BlockSpec((NC, BLK, D), lambda i: (0, i, 0)),
                  pl.BlockSpec((BLK, D), lambda i: (i, 0)),
                  pl.BlockSpec((BLK, D), lambda i: (i, 0)),
                  pl.BlockSpec((1, D), lambda i: (0, 0))],
        out_specs=pl.BlockSpec((BLK, D), lambda i: (i, 0)),
        out_shape=jax.ShapeDtypeStruct((N, D), jnp.float32),
    )(p, dis, h2, b2)


def kernel(x, edge_index, edge_weight, W1, b1, W2, b2):
    src = edge_index[0].astype(jnp.int32)
    dst = edge_index[1].astype(jnp.int32)
    ew = edge_weight.astype(jnp.float32)
    pad = EP - E
    src = jnp.concatenate([src, jnp.zeros((pad,), jnp.int32)])
    dst = jnp.concatenate([dst, jnp.zeros((pad,), jnp.int32)])
    ew = jnp.concatenate([ew, jnp.zeros((pad,), jnp.float32)])
    b1 = b1.reshape(1, D)
    b2 = b2.reshape(1, D)

    degp = _deg_kernel(dst, ew)
    h1 = _mm(x, W1)
    dis, h1p = _prep(degp, h1)
    p1 = _agg_kernel(h1p, src, dst, ew)
    h2, h2p = _mid(p1, dis, h1, b1, W2)
    p2 = _agg_kernel(h2p, src, dst, ew)
    return _fin(p2, dis, h2, b2)


# speedup vs baseline: 8.4387x; 1.3257x over previous
import dataclasses
import functools

import jax
import jax.numpy as jnp
from jax import lax
from jax.experimental import pallas as pl
from jax.experimental.pallas import tpu as pltpu
from jax.experimental.pallas import tpu_sc as plsc

N = 10000
D = 128
E = 320000

NC = 2
NS = 16
NW = NC * NS
CHUNK = 128
EPW = 10240
NCHUNK = EPW // CHUNK
EP = NW * EPW
NP = 10240
RPS = NP // NS

BLK = 1000

_mesh = plsc.VectorSubcoreMesh(core_axis_name="c", subcore_axis_name="s")

_sc_params = pltpu.CompilerParams()
if "needs_layout_passes" in pltpu.CompilerParams.__dataclass_fields__:
    _sc_params = dataclasses.replace(_sc_params, needs_layout_passes=False)

_GATHER_DNUMS = lax.GatherDimensionNumbers(
    offset_dims=(), collapsed_slice_dims=(0,), start_index_map=(0,))


def _splat(vec16, j):
    idx = jnp.full((16, 1), j, jnp.int32)
    return lax.gather(vec16, idx, _GATHER_DNUMS, slice_sizes=(1,),
                      mode=lax.GatherScatterMode.PROMISE_IN_BOUNDS)


@functools.partial(
    pl.kernel,
    out_type=jax.ShapeDtypeStruct((NC, NP, D), jnp.float32),
    mesh=_mesh,
    compiler_params=_sc_params,
    scratch_types=[
        pltpu.VMEM_SHARED((NP, D), jnp.float32),
        pltpu.VMEM((CHUNK,), jnp.int32),
        pltpu.VMEM((CHUNK,), jnp.float32),
        pltpu.VMEM((CHUNK, D), jnp.float32),
    ],
)
def _deg_kernel(dst_hbm, ew_hbm, out_hbm, deg_sh, dst_v, ew_v, bc_v):
    c = lax.axis_index("c")
    s = lax.axis_index("s")
    wid = c * NS + s
    zero16 = jnp.zeros((16,), jnp.float32)

    @pl.loop(0, CHUNK)
    def _(r):
        for q in range(D // 16):
            bc_v.at[r, pl.ds(q * 16, 16)][...] = zero16

    base = s * RPS

    @pl.loop(0, RPS // CHUNK)
    def _(t):
        pltpu.sync_copy(bc_v, deg_sh.at[pl.ds(base + t * CHUNK, CHUNK)])
    plsc.subcore_barrier()

    @pl.loop(0, NCHUNK)
    def _(k):
        ebase = wid * EPW + k * CHUNK
        pltpu.sync_copy(dst_hbm.at[pl.ds(ebase, CHUNK)], dst_v)
        pltpu.sync_copy(ew_hbm.at[pl.ds(ebase, CHUNK)], ew_v)

        @pl.loop(0, CHUNK // 16)
        def _(g):
            ew16 = ew_v[pl.ds(pl.multiple_of(g * 16, 16), 16)]
            for j in range(16):
                bc_v.at[g * 16 + j, pl.ds(0, 16)][...] = _splat(ew16, j)

        pltpu.sync_copy(bc_v, deg_sh.at[dst_v], add=True)

    plsc.subcore_barrier()
    pltpu.sync_copy(deg_sh.at[pl.ds(base, RPS)],
                    out_hbm.at[c, pl.ds(base, RPS)])


@functools.partial(
    pl.kernel,
    out_type=jax.ShapeDtypeStruct((NC, NP, D), jnp.float32),
    mesh=_mesh,
    compiler_params=_sc_params,
    scratch_types=[
        pltpu.VMEM_SHARED((NP, D), jnp.float32),
        pltpu.VMEM((CHUNK,), jnp.int32),
        pltpu.VMEM((CHUNK,), jnp.int32),
        pltpu.VMEM((CHUNK,), jnp.float32),
        pltpu.VMEM((CHUNK,), jnp.int32),
        pltpu.VMEM((CHUNK,), jnp.int32),
        pltpu.VMEM((CHUNK,), jnp.float32),
        pltpu.VMEM((CHUNK, D), jnp.float32),
        pltpu.VMEM((CHUNK, D), jnp.float32),
        pltpu.SemaphoreType.DMA,
        pltpu.SemaphoreType.DMA,
        pltpu.SemaphoreType.DMA,
        pltpu.SemaphoreType.DMA,
    ],
)
def _agg_kernel(h_hbm, src_hbm, dst_hbm, ew_hbm, out_hbm, acc_sh,
                src_a, dst_a, ew_a, src_b, dst_b, ew_b, rows_a, rows_b,
                sem_ra, sem_rb, sem_ia, sem_ib):
    c = lax.axis_index("c")
    s = lax.axis_index("s")
    wid = c * NS + s
    ebase0 = wid * EPW
    zero16 = jnp.zeros((16,), jnp.float32)

    @pl.loop(0, CHUNK)
    def _(r):
        for q in range(D // 16):
            rows_a.at[r, pl.ds(q * 16, 16)][...] = zero16

    base = s * RPS

    @pl.loop(0, RPS // CHUNK)
    def _(t):
        pltpu.sync_copy(rows_a, acc_sh.at[pl.ds(base + t * CHUNK, CHUNK)])
    plsc.subcore_barrier()

    def idx_fetch(kd, sv, dv, ev, sem):
        eb = ebase0 + kd * CHUNK
        pltpu.async_copy(src_hbm.at[pl.ds(eb, CHUNK)], sv, sem)
        pltpu.async_copy(dst_hbm.at[pl.ds(eb, CHUNK)], dv, sem)
        pltpu.async_copy(ew_hbm.at[pl.ds(eb, CHUNK)], ev, sem)

    def idx_wait(kd, sv, dv, ev, sem):
        eb = ebase0 + kd * CHUNK
        pltpu.make_async_copy(src_hbm.at[pl.ds(eb, CHUNK)], sv, sem).wait()
        pltpu.make_async_copy(dst_hbm.at[pl.ds(eb, CHUNK)], dv, sem).wait()
        pltpu.make_async_copy(ew_hbm.at[pl.ds(eb, CHUNK)], ev, sem).wait()

    idx_fetch(0, src_a, dst_a, ew_a, sem_ia)
    idx_wait(0, src_a, dst_a, ew_a, sem_ia)
    pltpu.async_copy(h_hbm.at[src_a], rows_a, sem_ra)
    idx_fetch(1, src_b, dst_b, ew_b, sem_ib)

    def halfstep(kd, sv, dv, ev, rows, sem_r, semi,
                 sv_o, dv_o, ev_o, rows_o, sem_ro, semi_o):
        @pl.when(kd + 1 < NCHUNK)
        def _():
            idx_wait(kd + 1, sv_o, dv_o, ev_o, semi_o)
            pltpu.async_copy(h_hbm.at[sv_o], rows_o, sem_ro)
        pltpu.make_async_copy(h_hbm.at[sv], rows, sem_r).wait()

        @plsc.parallel_loop(0, CHUNK // 16)
        def _(g):
            ew16 = ev[pl.ds(pl.multiple_of(g * 16, 16), 16)]
            for j in range(16):
                sp = _splat(ew16, j)
                r = g * 16 + j
                slots = [rows.at[r, pl.ds(q * 16, 16)] for q in range(D // 16)]
                vals = [sl[...] for sl in slots]
                vals = [v * sp for v in vals]
                for sl, v in zip(slots, vals):
                    sl[...] = v

        pltpu.sync_copy(rows, acc_sh.at[dv], add=True)

        @pl.when(kd + 2 < NCHUNK)
        def _():
            idx_fetch(kd + 2, sv, dv, ev, semi)

    @pl.loop(0, NCHUNK // 2)
    def _(i):
        k = i * 2
        halfstep(k, src_a, dst_a, ew_a, rows_a, sem_ra, sem_ia,
                 src_b, dst_b, ew_b, rows_b, sem_rb, sem_ib)
        halfstep(k + 1, src_b, dst_b, ew_b, rows_b, sem_rb, sem_ib,
                 src_a, dst_a, ew_a, rows_a, sem_ra, sem_ia)

    plsc.subcore_barrier()
    pltpu.sync_copy(acc_sh.at[pl.ds(base, RPS)],
                    out_hbm.at[c, pl.ds(base, RPS)])


def _mm_body(x_ref, w_ref, o_ref):
    o_ref[...] = jnp.dot(x_ref[...], w_ref[...],
                         preferred_element_type=jnp.float32)


def _mm(x, w):
    return pl.pallas_call(
        _mm_body,
        grid=(N // BLK,),
        in_specs=[pl.BlockSpec((BLK, D), lambda i: (i, 0)),
                  pl.BlockSpec((D, D), lambda i: (0, 0))],
        out_specs=pl.BlockSpec((BLK, D), lambda i: (i, 0)),
        out_shape=jax.ShapeDtypeStruct((N, D), jnp.float32),
    )(x, w)


def _prep_body(degp_ref, h1_ref, dis_ref, h1p_ref):
    degc = degp_ref[0, :, :1] + degp_ref[1, :, :1] + 1.0
    disc = lax.rsqrt(degc)
    dis = jnp.broadcast_to(disc, (BLK, D))
    dis_ref[...] = dis
    h1p_ref[...] = h1_ref[...] * dis


def _prep(degp, h1):
    return pl.pallas_call(
        _prep_body,
        grid=(N // BLK,),
        in_specs=[pl.BlockSpec((NC, BLK, D), lambda i: (0, i, 0)),
                  pl.BlockSpec((BLK, D), lambda i: (i, 0))],
        out_specs=[pl.BlockSpec((BLK, D), lambda i: (i, 0)),
                   pl.BlockSpec((BLK, D), lambda i: (i, 0))],
        out_shape=[jax.ShapeDtypeStruct((N, D), jnp.float32),
                   jax.ShapeDtypeStruct((N, D), jnp.float32)],
    )(degp, h1)


def _mid_body(p_ref, dis_ref, h1_ref, b_ref, w_ref, h2_ref, h2p_ref):
    dis = dis_ref[...]
    o1 = dis * (p_ref[0] + p_ref[1]) + dis * dis * h1_ref[...] + b_ref[...]
    r = jnp.maximum(o1, 0.0)
    h2 = jnp.dot(r, w_ref[...], preferred_element_type=jnp.float32)
    h2_ref[...] = h2
    h2p_ref[...] = h2 * dis


def _mid(p, dis, h1, b1, w2):
    return pl.pallas_call(
        _mid_body,
        grid=(N // BLK,),
        in_specs=[pl.BlockSpec((NC, BLK, D), lambda i: (0, i, 0)),
                  pl.BlockSpec((BLK, D), lambda i: (i, 0)),
                  pl.BlockSpec((BLK, D), lambda i: (i, 0)),
                  pl.BlockSpec((1, D), lambda i: (0, 0)),
                  pl.BlockSpec((D, D), lambda i: (0, 0))],
        out_specs=[pl.BlockSpec((BLK, D), lambda i: (i, 0)),
                   pl.BlockSpec((BLK, D), lambda i: (i, 0))],
        out_shape=[jax.ShapeDtypeStruct((N, D), jnp.float32),
                   jax.ShapeDtypeStruct((N, D), jnp.float32)],
    )(p, dis, h1, b1, w2)


def _fin_body(p_ref, dis_ref, h2_ref, b_ref, o_ref):
    dis = dis_ref[...]
    o2 = dis * (p_ref[0] + p_ref[1]) + dis * dis * h2_ref[...] + b_ref[...]
    nrm = jnp.sqrt(jnp.sum(o2 * o2, axis=1, keepdims=True))
    o_ref[...] = o2 / jnp.maximum(nrm, 1e-12)


def _fin(p, dis, h2, b2):
    return pl.pallas_call(
        _fin_body,
        grid=(N // BLK,),
        in_specs=[pl.BlockSpec((NC, BLK, D), lambda i: (0, i, 0)),
                  pl.BlockSpec((BLK, D), lambda i: (i, 0)),
                  pl.BlockSpec((BLK, D), lambda i: (i, 0)),
                  pl.BlockSpec((1, D), lambda i: (0, 0))],
        out_specs=pl.BlockSpec((BLK, D), lambda i: (i, 0)),
        out_shape=jax.ShapeDtypeStruct((N, D), jnp.float32),
    )(p, dis, h2, b2)


def kernel(x, edge_index, edge_weight, W1, b1, W2, b2):
    src = edge_index[0].astype(jnp.int32)
    dst = edge_index[1].astype(jnp.int32)
    ew = edge_weight.astype(jnp.float32)
    pad = EP - E
    src = jnp.concatenate([src, jnp.zeros((pad,), jnp.int32)])
    dst = jnp.concatenate([dst, jnp.zeros((pad,), jnp.int32)])
    ew = jnp.concatenate([ew, jnp.zeros((pad,), jnp.float32)])
    b1 = b1.reshape(1, D)
    b2 = b2.reshape(1, D)

    degp = _deg_kernel(dst, ew)
    h1 = _mm(x, W1)
    dis, h1p = _prep(degp, h1)
    p1 = _agg_kernel(h1p, src, dst, ew)
    h2, h2p = _mid(p1, dis, h1, b1, W2)
    p2 = _agg_kernel(h2p, src, dst, ew)
    return _fin(p2, dis, h2, b2)
